# baseline (speedup 1.0000x reference)
import functools

import jax
import jax.numpy as jnp
from jax import lax
from jax.experimental import pallas as pl
from jax.experimental.pallas import tpu as pltpu
from jax.experimental.pallas import tpu_sc as plsc

NN = 10000
EE = 320000
D = 128
LAT = 64
NG = 64
NW = 32
C = 128
G = 80
EP = NW * G * C
NC_AGG = 1
DUM = NN
NPAD = 10240
DEGW = 128
R = 1000


def _stripe_out(sid, cid, src_s, out_hbm):
    s_full = 632
    s_last = NN - 15 * s_full

    @pl.when(sid < 15)
    def _():
        pltpu.sync_copy(src_s.at[pl.ds(sid * s_full, s_full)],
                        out_hbm.at[cid, pl.ds(sid * s_full, s_full)])

    @pl.when(sid == 15)
    def _():
        pltpu.sync_copy(src_s.at[pl.ds(15 * s_full, s_last)],
                        out_hbm.at[cid, pl.ds(15 * s_full, s_last)])


def _make_sc_agg(nc):
    mesh = plsc.VectorSubcoreMesh(core_axis_name="c", subcore_axis_name="s",
                                  num_cores=nc)

    G2 = 40
    g_w = EP // (16 * nc * C)
    phases = g_w // G2

    def body(x_hbm, src_hbm, dst_hbm, agg_out, src_v, dst_v, rows_v, agg_s,
             s0, s1):
        cid = lax.axis_index("c")
        sid = lax.axis_index("s")
        wid = sid * nc + cid

        zv = jnp.zeros((16,), jnp.float32)
        for i in range(16):
            for j in range(D // 16):
                rows_v[0, i, pl.ds(j * 16, 16)] = zv
        zsrc = rows_v.at[0, pl.ds(0, 16)]

        rps = NPAD // 16

        @pl.loop(0, rps // 16)
        def _(k):
            pltpu.sync_copy(zsrc, agg_s.at[pl.ds(sid * rps + k * 16, 16)])

        plsc.subcore_barrier()

        def start(g, b, sem):
            pltpu.async_copy(x_hbm.at[src_v.at[g]], rows_v.at[b], sem)

        def wait(b, sem):
            pltpu.make_async_copy(x_hbm.at[src_v.at[0]], rows_v.at[b], sem).wait()

        def scat(g, b):
            pltpu.sync_copy(rows_v.at[b], agg_s.at[dst_v.at[g]], add=True)

        for phase in range(phases):
            pltpu.sync_copy(src_hbm.at[wid, pl.ds(phase * G2, G2)], src_v)
            pltpu.sync_copy(dst_hbm.at[wid, pl.ds(phase * G2, G2)], dst_v)

            start(0, 0, s0)

            @pl.loop(0, G2 // 2 - 1)
            def _(i):
                g = 2 * i
                start(g + 1, 1, s1)
                wait(0, s0)
                scat(g, 0)
                start(g + 2, 0, s0)
                wait(1, s1)
                scat(g + 1, 1)

            start(G2 - 1, 1, s1)
            wait(0, s0)
            scat(G2 - 2, 0)
            wait(1, s1)
            scat(G2 - 1, 1)

        plsc.subcore_barrier()
        _stripe_out(sid, cid, agg_s, agg_out)

    return pl.kernel(
        body,
        out_type=[jax.ShapeDtypeStruct((nc, NN, D), jnp.float32)],
        mesh=mesh,
        scratch_types=[
            pltpu.VMEM((G2, C), jnp.int32),
            pltpu.VMEM((G2, C), jnp.int32),
            pltpu.VMEM((2, C, D), jnp.float32),
            pltpu.VMEM_SHARED((NPAD, D), jnp.float32),
            pltpu.SemaphoreType.DMA,
            pltpu.SemaphoreType.DMA,
        ])


def _make_sc_deg():
    mesh = plsc.VectorSubcoreMesh(core_axis_name="c", subcore_axis_name="s")

    def body(dst_hbm, deg_out, dst_v, ones_v, zdeg_v, deg_s):
        cid = lax.axis_index("c")
        sid = lax.axis_index("s")
        wid = sid * 2 + cid

        zv = jnp.zeros((16,), jnp.float32)
        ov = jnp.ones((16,), jnp.float32)
        for i in range(C):
            for j in range(DEGW // 16):
                ones_v[i, pl.ds(j * 16, 16)] = ov
        for i in range(16):
            for j in range(DEGW // 16):
                zdeg_v[i, pl.ds(j * 16, 16)] = zv

        rps = NPAD // 16

        @pl.loop(0, rps // 16)
        def _(k):
            pltpu.sync_copy(zdeg_v, deg_s.at[pl.ds(sid * rps + k * 16, 16)])

        plsc.subcore_barrier()
        pltpu.sync_copy(dst_hbm.at[wid], dst_v)

        @pl.loop(0, G)
        def _(g):
            pltpu.sync_copy(ones_v, deg_s.at[dst_v.at[g]], add=True)

        plsc.subcore_barrier()
        _stripe_out(sid, cid, deg_s, deg_out)

    return pl.kernel(
        body,
        out_type=[jax.ShapeDtypeStruct((2, NN, DEGW), jnp.float32)],
        mesh=mesh,
        scratch_types=[
            pltpu.VMEM((G, C), jnp.int32),
            pltpu.VMEM((C, DEGW), jnp.float32),
            pltpu.VMEM((16, DEGW), jnp.float32),
            pltpu.VMEM_SHARED((NPAD, DEGW), jnp.float32),
        ])


_sc_agg = _make_sc_agg(NC_AGG)
_sc_deg = _make_sc_deg()


def _tc_layer1(x, aggp, degp, w_self, w_nei, b):
    npart = aggp.shape[0]

    def body(x_ref, aggp_ref, degp_ref, ws_ref, wn_ref, b_ref, o_ref):
        agg = aggp_ref[0]
        for p in range(1, npart):
            agg += aggp_ref[p]
        deg = jnp.maximum(degp_ref[0, :, 0:1] + degp_ref[1, :, 0:1], 1.0)
        h = jnp.dot(x_ref[...], ws_ref[...], preferred_element_type=jnp.float32)
        h += jnp.dot(agg / deg, wn_ref[...], preferred_element_type=jnp.float32)
        h += b_ref[...]
        o_ref[...] = jnp.maximum(h, 0.0)

    return pl.pallas_call(
        body,
        grid=(NN // R,),
        in_specs=[
            pl.BlockSpec((R, D), lambda i: (i, 0)),
            pl.BlockSpec((npart, R, D), lambda i: (0, i, 0)),
            pl.BlockSpec((2, R, DEGW), lambda i: (0, i, 0)),
            pl.BlockSpec((D, D), lambda i: (0, 0)),
            pl.BlockSpec((D, D), lambda i: (0, 0)),
            pl.BlockSpec((1, D), lambda i: (0, 0)),
        ],
        out_specs=pl.BlockSpec((R, D), lambda i: (i, 0)),
        out_shape=jax.ShapeDtypeStruct((NN, D), jnp.float32),
    )(x, aggp, degp, w_self, w_nei, b)


def _tc_layer2_pool(h1, aggp, degp, w_self, w_nei, b, bat3, w_lat, b_lat):
    nb = NN // R
    npart = aggp.shape[0]

    def body(h_ref, aggp_ref, degp_ref, ws_ref, wn_ref, b_ref, bat_ref,
             wl_ref, bl_ref, z_ref, pool_acc, cnt_acc):
        i = pl.program_id(0)

        @pl.when(i == 0)
        def _():
            pool_acc[...] = jnp.zeros_like(pool_acc)
            cnt_acc[...] = jnp.zeros_like(cnt_acc)

        agg = aggp_ref[0]
        for p in range(1, npart):
            agg += aggp_ref[p]
        deg = jnp.maximum(degp_ref[0, :, 0:1] + degp_ref[1, :, 0:1], 1.0)
        h2 = jnp.dot(h_ref[...], ws_ref[...], preferred_element_type=jnp.float32)
        h2 += jnp.dot(agg / deg, wn_ref[...], preferred_element_type=jnp.float32)
        h2 += b_ref[...]
        h2 = jnp.maximum(h2, 0.0)

        ids = bat_ref[0, 0, :]
        rid = lax.broadcasted_iota(jnp.int32, (NG, R), 0)
        onehot = (rid == ids[None, :]).astype(jnp.float32)
        pool_acc[...] += jnp.dot(onehot, h2, preferred_element_type=jnp.float32)
        cnt_acc[...] += jnp.broadcast_to(
            jnp.sum(onehot, axis=1, keepdims=True), (NG, D))

        @pl.when(i == nb - 1)
        def _():
            pooled = pool_acc[...] / jnp.maximum(cnt_acc[...], 1.0)
            z_ref[...] = jnp.dot(pooled, wl_ref[...],
                                 preferred_element_type=jnp.float32) + bl_ref[...]

    return pl.pallas_call(
        body,
        grid=(nb,),
        in_specs=[
            pl.BlockSpec((R, D), lambda i: (i, 0)),
            pl.BlockSpec((npart, R, D), lambda i: (0, i, 0)),
            pl.BlockSpec((2, R, DEGW), lambda i: (0, i, 0)),
            pl.BlockSpec((D, D), lambda i: (0, 0)),
            pl.BlockSpec((D, D), lambda i: (0, 0)),
            pl.BlockSpec((1, D), lambda i: (0, 0)),
            pl.BlockSpec((1, 1, R), lambda i: (i, 0, 0)),
            pl.BlockSpec((D, LAT), lambda i: (0, 0)),
            pl.BlockSpec((1, LAT), lambda i: (0, 0)),
        ],
        out_specs=pl.BlockSpec((NG, LAT), lambda i: (0, 0)),
        out_shape=jax.ShapeDtypeStruct((NG, LAT), jnp.float32),
        scratch_shapes=[
            pltpu.VMEM((NG, D), jnp.float32),
            pltpu.VMEM((NG, D), jnp.float32),
        ],
    )(h1, aggp, degp, w_self, w_nei, b, bat3, w_lat, b_lat)


def kernel(x, edge_index, batch, W1_self, W1_nei, b1, W2_self, W2_nei, b2,
           W_lat, b_lat):
    src = edge_index[0].astype(jnp.int32)
    dst = edge_index[1].astype(jnp.int32)
    pad = EP - EE
    srcf = jnp.concatenate([src, jnp.zeros((pad,), jnp.int32)])
    dstf = jnp.concatenate([dst, jnp.full((pad,), DUM, jnp.int32)])
    nwa = 16 * NC_AGG
    srcp = srcf.reshape(nwa, EP // (nwa * C), C)
    dstp = dstf.reshape(nwa, EP // (nwa * C), C)
    dstp_deg = dstf.reshape(NW, G, C)
    bat3 = batch.astype(jnp.int32).reshape(NN // R, 1, R)

    (degp,) = _sc_deg(dstp_deg)
    (agg1,) = _sc_agg(x, srcp, dstp)
    h1 = _tc_layer1(x, agg1, degp, W1_self, W1_nei, b1.reshape(1, D))
    (agg2,) = _sc_agg(h1, srcp, dstp)
    z = _tc_layer2_pool(h1, agg2, degp, W2_self, W2_nei, b2.reshape(1, D),
                        bat3, W_lat, b_lat.reshape(1, LAT))
    return z


# speedup vs baseline: 1.1239x; 1.1239x over previous
import functools

import jax
import jax.numpy as jnp
from jax import lax
from jax.experimental import pallas as pl
from jax.experimental.pallas import tpu as pltpu
from jax.experimental.pallas import tpu_sc as plsc

NN = 10000
EE = 320000
D = 128
LAT = 64
NG = 64
NW = 32
C = 128
G = 80
EP = NW * G * C
DUM = NN
NPAD = 10240
DEGW = 128
R = 1000


def _stripe_out(sid, cid, src_s, out_hbm):
    s_full = 632
    s_last = NN - 15 * s_full

    @pl.when(sid < 15)
    def _():
        pltpu.sync_copy(src_s.at[pl.ds(sid * s_full, s_full)],
                        out_hbm.at[cid, pl.ds(sid * s_full, s_full)])

    @pl.when(sid == 15)
    def _():
        pltpu.sync_copy(src_s.at[pl.ds(15 * s_full, s_last)],
                        out_hbm.at[cid, pl.ds(15 * s_full, s_last)])


CG = 64
NBUF = 4
GW = EP // (NW * CG)
G2 = 40
PHASES = GW // G2


def _make_sc_agg():
    mesh = plsc.VectorSubcoreMesh(core_axis_name="c", subcore_axis_name="s")

    def body(x_hbm, src_hbm, dst_hbm, agg_out, src_v, dst_v, rows_v, agg_s,
             *sems):
        cid = lax.axis_index("c")
        sid = lax.axis_index("s")
        wid = sid * 2 + cid

        zv = jnp.zeros((16,), jnp.float32)
        for i in range(16):
            for j in range(D // 16):
                rows_v[0, i, pl.ds(j * 16, 16)] = zv
        zsrc = rows_v.at[0, pl.ds(0, 16)]

        rps = NPAD // 16

        @pl.loop(0, rps // 16)
        def _(k):
            pltpu.sync_copy(zsrc, agg_s.at[pl.ds(sid * rps + k * 16, 16)])

        plsc.subcore_barrier()

        def start(g, b):
            pltpu.async_copy(x_hbm.at[src_v.at[g]], rows_v.at[b], sems[b])

        def wait(b):
            pltpu.make_async_copy(x_hbm.at[src_v.at[0]], rows_v.at[b],
                                  sems[b]).wait()

        def scat(g, b):
            pltpu.sync_copy(rows_v.at[b], agg_s.at[dst_v.at[g]], add=True)

        for phase in range(PHASES):
            pltpu.sync_copy(src_hbm.at[wid, pl.ds(phase * G2, G2)], src_v)
            pltpu.sync_copy(dst_hbm.at[wid, pl.ds(phase * G2, G2)], dst_v)

            for b in range(NBUF):
                start(b, b)

            @pl.loop(0, (G2 - NBUF) // NBUF)
            def _(i):
                g0 = i * NBUF
                for b in range(NBUF):
                    wait(b)
                    scat(g0 + b, b)
                    start(g0 + b + NBUF, b)

            for b in range(NBUF):
                wait(b)
                scat(G2 - NBUF + b, b)

        plsc.subcore_barrier()
        _stripe_out(sid, cid, agg_s, agg_out)

    return pl.kernel(
        body,
        out_type=[jax.ShapeDtypeStruct((2, NN, D), jnp.float32)],
        mesh=mesh,
        scratch_types=[
            pltpu.VMEM((G2, CG), jnp.int32),
            pltpu.VMEM((G2, CG), jnp.int32),
            pltpu.VMEM((NBUF, CG, D), jnp.float32),
            pltpu.VMEM_SHARED((NPAD, D), jnp.float32),
        ] + [pltpu.SemaphoreType.DMA] * NBUF)


def _make_sc_deg():
    mesh = plsc.VectorSubcoreMesh(core_axis_name="c", subcore_axis_name="s")

    def body(dst_hbm, deg_out, dst_v, ones_v, zdeg_v, deg_s):
        cid = lax.axis_index("c")
        sid = lax.axis_index("s")
        wid = sid * 2 + cid

        zv = jnp.zeros((16,), jnp.float32)
        ov = jnp.ones((16,), jnp.float32)
        for i in range(C):
            for j in range(DEGW // 16):
                ones_v[i, pl.ds(j * 16, 16)] = ov
        for i in range(16):
            for j in range(DEGW // 16):
                zdeg_v[i, pl.ds(j * 16, 16)] = zv

        rps = NPAD // 16

        @pl.loop(0, rps // 16)
        def _(k):
            pltpu.sync_copy(zdeg_v, deg_s.at[pl.ds(sid * rps + k * 16, 16)])

        plsc.subcore_barrier()
        pltpu.sync_copy(dst_hbm.at[wid], dst_v)

        @pl.loop(0, G)
        def _(g):
            pltpu.sync_copy(ones_v, deg_s.at[dst_v.at[g]], add=True)

        plsc.subcore_barrier()
        _stripe_out(sid, cid, deg_s, deg_out)

    return pl.kernel(
        body,
        out_type=[jax.ShapeDtypeStruct((2, NN, DEGW), jnp.float32)],
        mesh=mesh,
        scratch_types=[
            pltpu.VMEM((G, C), jnp.int32),
            pltpu.VMEM((C, DEGW), jnp.float32),
            pltpu.VMEM((16, DEGW), jnp.float32),
            pltpu.VMEM_SHARED((NPAD, DEGW), jnp.float32),
        ])


_sc_agg = _make_sc_agg()
_sc_deg = _make_sc_deg()


def _tc_layer1(x, aggp, degp, w_self, w_nei, b):
    def body(x_ref, aggp_ref, degp_ref, ws_ref, wn_ref, b_ref, o_ref):
        agg = aggp_ref[0] + aggp_ref[1]
        deg = jnp.maximum(degp_ref[0, :, 0:1] + degp_ref[1, :, 0:1], 1.0)
        h = jnp.dot(x_ref[...], ws_ref[...], preferred_element_type=jnp.float32)
        h += jnp.dot(agg / deg, wn_ref[...], preferred_element_type=jnp.float32)
        h += b_ref[...]
        o_ref[...] = jnp.maximum(h, 0.0)

    return pl.pallas_call(
        body,
        grid=(NN // R,),
        in_specs=[
            pl.BlockSpec((R, D), lambda i: (i, 0)),
            pl.BlockSpec((2, R, D), lambda i: (0, i, 0)),
            pl.BlockSpec((2, R, DEGW), lambda i: (0, i, 0)),
            pl.BlockSpec((D, D), lambda i: (0, 0)),
            pl.BlockSpec((D, D), lambda i: (0, 0)),
            pl.BlockSpec((1, D), lambda i: (0, 0)),
        ],
        out_specs=pl.BlockSpec((R, D), lambda i: (i, 0)),
        out_shape=jax.ShapeDtypeStruct((NN, D), jnp.float32),
    )(x, aggp, degp, w_self, w_nei, b)


def _tc_layer2_pool(h1, aggp, degp, w_self, w_nei, b, bat3, w_lat, b_lat):
    nb = NN // R

    def body(h_ref, aggp_ref, degp_ref, ws_ref, wn_ref, b_ref, bat_ref,
             wl_ref, bl_ref, z_ref, pool_acc, cnt_acc):
        i = pl.program_id(0)

        @pl.when(i == 0)
        def _():
            pool_acc[...] = jnp.zeros_like(pool_acc)
            cnt_acc[...] = jnp.zeros_like(cnt_acc)

        agg = aggp_ref[0] + aggp_ref[1]
        deg = jnp.maximum(degp_ref[0, :, 0:1] + degp_ref[1, :, 0:1], 1.0)
        h2 = jnp.dot(h_ref[...], ws_ref[...], preferred_element_type=jnp.float32)
        h2 += jnp.dot(agg / deg, wn_ref[...], preferred_element_type=jnp.float32)
        h2 += b_ref[...]
        h2 = jnp.maximum(h2, 0.0)

        ids = bat_ref[0, 0, :]
        rid = lax.broadcasted_iota(jnp.int32, (NG, R), 0)
        onehot = (rid == ids[None, :]).astype(jnp.float32)
        pool_acc[...] += jnp.dot(onehot, h2, preferred_element_type=jnp.float32)
        cnt_acc[...] += jnp.broadcast_to(
            jnp.sum(onehot, axis=1, keepdims=True), (NG, D))

        @pl.when(i == nb - 1)
        def _():
            pooled = pool_acc[...] / jnp.maximum(cnt_acc[...], 1.0)
            z_ref[...] = jnp.dot(pooled, wl_ref[...],
                                 preferred_element_type=jnp.float32) + bl_ref[...]

    return pl.pallas_call(
        body,
        grid=(nb,),
        in_specs=[
            pl.BlockSpec((R, D), lambda i: (i, 0)),
            pl.BlockSpec((2, R, D), lambda i: (0, i, 0)),
            pl.BlockSpec((2, R, DEGW), lambda i: (0, i, 0)),
            pl.BlockSpec((D, D), lambda i: (0, 0)),
            pl.BlockSpec((D, D), lambda i: (0, 0)),
            pl.BlockSpec((1, D), lambda i: (0, 0)),
            pl.BlockSpec((1, 1, R), lambda i: (i, 0, 0)),
            pl.BlockSpec((D, LAT), lambda i: (0, 0)),
            pl.BlockSpec((1, LAT), lambda i: (0, 0)),
        ],
        out_specs=pl.BlockSpec((NG, LAT), lambda i: (0, 0)),
        out_shape=jax.ShapeDtypeStruct((NG, LAT), jnp.float32),
        scratch_shapes=[
            pltpu.VMEM((NG, D), jnp.float32),
            pltpu.VMEM((NG, D), jnp.float32),
        ],
    )(h1, aggp, degp, w_self, w_nei, b, bat3, w_lat, b_lat)


def kernel(x, edge_index, batch, W1_self, W1_nei, b1, W2_self, W2_nei, b2,
           W_lat, b_lat):
    src = edge_index[0].astype(jnp.int32)
    dst = edge_index[1].astype(jnp.int32)
    pad = EP - EE
    srcf = jnp.concatenate([src, jnp.zeros((pad,), jnp.int32)])
    dstf = jnp.concatenate([dst, jnp.full((pad,), DUM, jnp.int32)])
    srcp = srcf.reshape(NW, GW, CG)
    dstp = dstf.reshape(NW, GW, CG)
    dstp_deg = dstf.reshape(NW, G, C)
    bat3 = batch.astype(jnp.int32).reshape(NN // R, 1, R)

    (degp,) = _sc_deg(dstp_deg)
    (agg1,) = _sc_agg(x, srcp, dstp)
    h1 = _tc_layer1(x, agg1, degp, W1_self, W1_nei, b1.reshape(1, D))
    (agg2,) = _sc_agg(h1, srcp, dstp)
    z = _tc_layer2_pool(h1, agg2, degp, W2_self, W2_nei, b2.reshape(1, D),
                        bat3, W_lat, b_lat.reshape(1, LAT))
    return z


# speedup vs baseline: 1.1997x; 1.0674x over previous
import functools

import jax
import jax.numpy as jnp
from jax import lax
from jax.experimental import pallas as pl
from jax.experimental.pallas import tpu as pltpu
from jax.experimental.pallas import tpu_sc as plsc

NN = 10000
EE = 320000
D = 128
LAT = 64
NG = 64
NW = 32
C = 128
G = 80
EP = NW * G * C
DUM = NN
NPAD = 10240
DEGW = 128
R = 1000


def _stripe_out(sid, cid, src_s, out_hbm):
    s_full = 632
    s_last = NN - 15 * s_full

    @pl.when(sid < 15)
    def _():
        pltpu.sync_copy(src_s.at[pl.ds(sid * s_full, s_full)],
                        out_hbm.at[cid, pl.ds(sid * s_full, s_full)])

    @pl.when(sid == 15)
    def _():
        pltpu.sync_copy(src_s.at[pl.ds(15 * s_full, s_last)],
                        out_hbm.at[cid, pl.ds(15 * s_full, s_last)])


CG = 64
NBUF = 4
G2 = 40
NGRP = EP // CG
N0 = 240
N1 = NGRP // 16 - N0


def _make_sc_agg():
    mesh = plsc.VectorSubcoreMesh(core_axis_name="c", subcore_axis_name="s")

    def body(x_hbm, src_hbm, dst_hbm, agg_out, src_v, dst_v, rows_v, agg_s,
             *sems):
        cid = lax.axis_index("c")
        sid = lax.axis_index("s")

        zv = jnp.zeros((16,), jnp.float32)
        for i in range(16):
            for j in range(D // 16):
                rows_v[0, i, pl.ds(j * 16, 16)] = zv
        zsrc = rows_v.at[0, pl.ds(0, 16)]

        rps = NPAD // 16

        @pl.loop(0, rps // 16)
        def _(k):
            pltpu.sync_copy(zsrc, agg_s.at[pl.ds(sid * rps + k * 16, 16)])

        plsc.subcore_barrier()

        def start(g, b):
            pltpu.async_copy(x_hbm.at[src_v.at[g]], rows_v.at[b], sems[b])

        def wait(b):
            pltpu.make_async_copy(x_hbm.at[src_v.at[0]], rows_v.at[b],
                                  sems[b]).wait()

        def scat(g, b):
            pltpu.sync_copy(rows_v.at[b], agg_s.at[dst_v.at[g]], add=True)

        def run_phase(gbase):
            pltpu.sync_copy(src_hbm.at[pl.ds(gbase, G2)], src_v)
            pltpu.sync_copy(dst_hbm.at[pl.ds(gbase, G2)], dst_v)

            for b in range(NBUF):
                start(b, b)

            @pl.loop(0, (G2 - NBUF) // NBUF)
            def _(i):
                g0 = i * NBUF
                for b in range(NBUF):
                    wait(b)
                    scat(g0 + b, b)
                    start(g0 + b + NBUF, b)

            for b in range(NBUF):
                wait(b)
                scat(G2 - NBUF + b, b)

        @pl.when(cid == 0)
        def _():
            for ph in range(N0 // G2):
                run_phase(sid * N0 + ph * G2)

        @pl.when(cid == 1)
        def _():
            for ph in range(N1 // G2):
                run_phase(16 * N0 + sid * N1 + ph * G2)

        plsc.subcore_barrier()
        _stripe_out(sid, cid, agg_s, agg_out)

    return pl.kernel(
        body,
        out_type=[jax.ShapeDtypeStruct((2, NN, D), jnp.float32)],
        mesh=mesh,
        scratch_types=[
            pltpu.VMEM((G2, CG), jnp.int32),
            pltpu.VMEM((G2, CG), jnp.int32),
            pltpu.VMEM((NBUF, CG, D), jnp.float32),
            pltpu.VMEM_SHARED((NPAD, D), jnp.float32),
        ] + [pltpu.SemaphoreType.DMA] * NBUF)


def _make_sc_deg():
    mesh = plsc.VectorSubcoreMesh(core_axis_name="c", subcore_axis_name="s")

    def body(dst_hbm, deg_out, dst_v, ones_v, zdeg_v, deg_s):
        cid = lax.axis_index("c")
        sid = lax.axis_index("s")
        wid = sid * 2 + cid

        zv = jnp.zeros((16,), jnp.float32)
        ov = jnp.ones((16,), jnp.float32)
        for i in range(C):
            for j in range(DEGW // 16):
                ones_v[i, pl.ds(j * 16, 16)] = ov
        for i in range(16):
            for j in range(DEGW // 16):
                zdeg_v[i, pl.ds(j * 16, 16)] = zv

        rps = NPAD // 16

        @pl.loop(0, rps // 16)
        def _(k):
            pltpu.sync_copy(zdeg_v, deg_s.at[pl.ds(sid * rps + k * 16, 16)])

        plsc.subcore_barrier()
        pltpu.sync_copy(dst_hbm.at[wid], dst_v)

        @pl.loop(0, G)
        def _(g):
            pltpu.sync_copy(ones_v, deg_s.at[dst_v.at[g]], add=True)

        plsc.subcore_barrier()
        _stripe_out(sid, cid, deg_s, deg_out)

    return pl.kernel(
        body,
        out_type=[jax.ShapeDtypeStruct((2, NN, DEGW), jnp.float32)],
        mesh=mesh,
        scratch_types=[
            pltpu.VMEM((G, C), jnp.int32),
            pltpu.VMEM((C, DEGW), jnp.float32),
            pltpu.VMEM((16, DEGW), jnp.float32),
            pltpu.VMEM_SHARED((NPAD, DEGW), jnp.float32),
        ])


_sc_agg = _make_sc_agg()
_sc_deg = _make_sc_deg()


def _tc_layer1(x, aggp, degp, w_self, w_nei, b):
    def body(x_ref, aggp_ref, degp_ref, ws_ref, wn_ref, b_ref, o_ref):
        agg = aggp_ref[0] + aggp_ref[1]
        deg = jnp.maximum(degp_ref[0, :, 0:1] + degp_ref[1, :, 0:1], 1.0)
        h = jnp.dot(x_ref[...], ws_ref[...], preferred_element_type=jnp.float32)
        h += jnp.dot(agg / deg, wn_ref[...], preferred_element_type=jnp.float32)
        h += b_ref[...]
        o_ref[...] = jnp.maximum(h, 0.0)

    return pl.pallas_call(
        body,
        grid=(NN // R,),
        in_specs=[
            pl.BlockSpec((R, D), lambda i: (i, 0)),
            pl.BlockSpec((2, R, D), lambda i: (0, i, 0)),
            pl.BlockSpec((2, R, DEGW), lambda i: (0, i, 0)),
            pl.BlockSpec((D, D), lambda i: (0, 0)),
            pl.BlockSpec((D, D), lambda i: (0, 0)),
            pl.BlockSpec((1, D), lambda i: (0, 0)),
        ],
        out_specs=pl.BlockSpec((R, D), lambda i: (i, 0)),
        out_shape=jax.ShapeDtypeStruct((NN, D), jnp.float32),
    )(x, aggp, degp, w_self, w_nei, b)


def _tc_layer2_pool(h1, aggp, degp, w_self, w_nei, b, bat3, w_lat, b_lat):
    nb = NN // R

    def body(h_ref, aggp_ref, degp_ref, ws_ref, wn_ref, b_ref, bat_ref,
             wl_ref, bl_ref, z_ref, pool_acc, cnt_acc):
        i = pl.program_id(0)

        @pl.when(i == 0)
        def _():
            pool_acc[...] = jnp.zeros_like(pool_acc)
            cnt_acc[...] = jnp.zeros_like(cnt_acc)

        agg = aggp_ref[0] + aggp_ref[1]
        deg = jnp.maximum(degp_ref[0, :, 0:1] + degp_ref[1, :, 0:1], 1.0)
        h2 = jnp.dot(h_ref[...], ws_ref[...], preferred_element_type=jnp.float32)
        h2 += jnp.dot(agg / deg, wn_ref[...], preferred_element_type=jnp.float32)
        h2 += b_ref[...]
        h2 = jnp.maximum(h2, 0.0)

        ids = bat_ref[0, 0, :]
        rid = lax.broadcasted_iota(jnp.int32, (NG, R), 0)
        onehot = (rid == ids[None, :]).astype(jnp.float32)
        pool_acc[...] += jnp.dot(onehot, h2, preferred_element_type=jnp.float32)
        cnt_acc[...] += jnp.broadcast_to(
            jnp.sum(onehot, axis=1, keepdims=True), (NG, D))

        @pl.when(i == nb - 1)
        def _():
            pooled = pool_acc[...] / jnp.maximum(cnt_acc[...], 1.0)
            z_ref[...] = jnp.dot(pooled, wl_ref[...],
                                 preferred_element_type=jnp.float32) + bl_ref[...]

    return pl.pallas_call(
        body,
        grid=(nb,),
        in_specs=[
            pl.BlockSpec((R, D), lambda i: (i, 0)),
            pl.BlockSpec((2, R, D), lambda i: (0, i, 0)),
            pl.BlockSpec((2, R, DEGW), lambda i: (0, i, 0)),
            pl.BlockSpec((D, D), lambda i: (0, 0)),
            pl.BlockSpec((D, D), lambda i: (0, 0)),
            pl.BlockSpec((1, D), lambda i: (0, 0)),
            pl.BlockSpec((1, 1, R), lambda i: (i, 0, 0)),
            pl.BlockSpec((D, LAT), lambda i: (0, 0)),
            pl.BlockSpec((1, LAT), lambda i: (0, 0)),
        ],
        out_specs=pl.BlockSpec((NG, LAT), lambda i: (0, 0)),
        out_shape=jax.ShapeDtypeStruct((NG, LAT), jnp.float32),
        scratch_shapes=[
            pltpu.VMEM((NG, D), jnp.float32),
            pltpu.VMEM((NG, D), jnp.float32),
        ],
    )(h1, aggp, degp, w_self, w_nei, b, bat3, w_lat, b_lat)


def kernel(x, edge_index, batch, W1_self, W1_nei, b1, W2_self, W2_nei, b2,
           W_lat, b_lat):
    src = edge_index[0].astype(jnp.int32)
    dst = edge_index[1].astype(jnp.int32)
    pad = EP - EE
    srcf = jnp.concatenate([src, jnp.zeros((pad,), jnp.int32)])
    dstf = jnp.concatenate([dst, jnp.full((pad,), DUM, jnp.int32)])
    srcp = srcf.reshape(NGRP, CG)
    dstp = dstf.reshape(NGRP, CG)
    dstp_deg = dstf.reshape(NW, G, C)
    bat3 = batch.astype(jnp.int32).reshape(NN // R, 1, R)

    (degp,) = _sc_deg(dstp_deg)
    (agg1,) = _sc_agg(x, srcp, dstp)
    h1 = _tc_layer1(x, agg1, degp, W1_self, W1_nei, b1.reshape(1, D))
    (agg2,) = _sc_agg(h1, srcp, dstp)
    z = _tc_layer2_pool(h1, agg2, degp, W2_self, W2_nei, b2.reshape(1, D),
                        bat3, W_lat, b_lat.reshape(1, LAT))
    return z


# speedup vs baseline: 1.2775x; 1.0648x over previous
import functools

import jax
import jax.numpy as jnp
from jax import lax
from jax.experimental import pallas as pl
from jax.experimental.pallas import tpu as pltpu
from jax.experimental.pallas import tpu_sc as plsc

NN = 10000
EE = 320000
D = 128
LAT = 64
NG = 64
NW = 32
C = 128
G = 80
EP = NW * G * C
DUM = NN
NPAD = 10240
DEGW = 128
R = 1000


def _stripe_out(sid, cid, src_s, out_hbm):
    s_full = 640
    s_last = NN - 15 * s_full

    @pl.when(sid < 15)
    def _():
        pltpu.sync_copy(src_s.at[pl.ds(sid * s_full, s_full)],
                        out_hbm.at[cid, pl.ds(sid * s_full, s_full)])

    @pl.when(sid == 15)
    def _():
        pltpu.sync_copy(src_s.at[pl.ds(15 * s_full, s_last)],
                        out_hbm.at[cid, pl.ds(15 * s_full, s_last)])


CG = 128
NBUF = 2
G2 = 40
NGRP = EP // CG
N0 = 120
N1 = NGRP // 16 - N0


def _make_sc_agg():
    mesh = plsc.VectorSubcoreMesh(core_axis_name="c", subcore_axis_name="s")

    def body(x_hbm, src_hbm, dst_hbm, agg_out, src_v, dst_v, rows_v, agg_s,
             *sems):
        cid = lax.axis_index("c")
        sid = lax.axis_index("s")

        zv = jnp.zeros((16,), jnp.float32)
        for i in range(16):
            for j in range(D // 16):
                rows_v[0, i, pl.ds(j * 16, 16)] = zv
        zsrc = rows_v.at[0, pl.ds(0, 16)]

        rps = NPAD // 16

        @pl.loop(0, rps // 16)
        def _(k):
            pltpu.sync_copy(zsrc, agg_s.at[pl.ds(sid * rps + k * 16, 16)])

        plsc.subcore_barrier()

        def start(g, b):
            pltpu.async_copy(x_hbm.at[src_v.at[g]], rows_v.at[b], sems[b])

        def wait(b):
            pltpu.make_async_copy(x_hbm.at[src_v.at[0]], rows_v.at[b],
                                  sems[b]).wait()

        def scat(g, b):
            pltpu.sync_copy(rows_v.at[b], agg_s.at[dst_v.at[g]], add=True)

        def run_phase(gbase):
            pltpu.sync_copy(src_hbm.at[pl.ds(gbase, G2)], src_v)
            pltpu.sync_copy(dst_hbm.at[pl.ds(gbase, G2)], dst_v)

            for b in range(NBUF):
                start(b, b)

            @pl.loop(0, (G2 - NBUF) // NBUF)
            def _(i):
                g0 = i * NBUF
                for b in range(NBUF):
                    wait(b)
                    scat(g0 + b, b)
                    start(g0 + b + NBUF, b)

            for b in range(NBUF):
                wait(b)
                scat(G2 - NBUF + b, b)

        @pl.when(cid == 0)
        def _():
            for ph in range(N0 // G2):
                run_phase(sid * N0 + ph * G2)

        @pl.when(cid == 1)
        def _():
            for ph in range(N1 // G2):
                run_phase(16 * N0 + sid * N1 + ph * G2)

        plsc.subcore_barrier()
        _stripe_out(sid, cid, agg_s, agg_out)

    return pl.kernel(
        body,
        out_type=[jax.ShapeDtypeStruct((2, NN, D), jnp.float32)],
        mesh=mesh,
        scratch_types=[
            pltpu.VMEM((G2, CG), jnp.int32),
            pltpu.VMEM((G2, CG), jnp.int32),
            pltpu.VMEM((NBUF, CG, D), jnp.float32),
            pltpu.VMEM_SHARED((NPAD, D), jnp.float32),
        ] + [pltpu.SemaphoreType.DMA] * NBUF)


def _make_sc_deg():
    mesh = plsc.VectorSubcoreMesh(core_axis_name="c", subcore_axis_name="s")

    def body(dst_hbm, deg_out, dst_v, ones_v, zdeg_v, deg_s):
        cid = lax.axis_index("c")
        sid = lax.axis_index("s")
        wid = sid * 2 + cid

        zv = jnp.zeros((16,), jnp.float32)
        ov = jnp.ones((16,), jnp.float32)
        for i in range(C):
            for j in range(DEGW // 16):
                ones_v[i, pl.ds(j * 16, 16)] = ov
        for i in range(16):
            for j in range(DEGW // 16):
                zdeg_v[i, pl.ds(j * 16, 16)] = zv

        rps = NPAD // 16

        @pl.loop(0, rps // 16)
        def _(k):
            pltpu.sync_copy(zdeg_v, deg_s.at[pl.ds(sid * rps + k * 16, 16)])

        plsc.subcore_barrier()
        pltpu.sync_copy(dst_hbm.at[wid], dst_v)

        @pl.loop(0, G)
        def _(g):
            pltpu.sync_copy(ones_v, deg_s.at[dst_v.at[g]], add=True)

        plsc.subcore_barrier()
        _stripe_out(sid, cid, deg_s, deg_out)

    return pl.kernel(
        body,
        out_type=[jax.ShapeDtypeStruct((2, NN, DEGW), jnp.float32)],
        mesh=mesh,
        scratch_types=[
            pltpu.VMEM((G, C), jnp.int32),
            pltpu.VMEM((C, DEGW), jnp.float32),
            pltpu.VMEM((16, DEGW), jnp.float32),
            pltpu.VMEM_SHARED((NPAD, DEGW), jnp.float32),
        ])


_sc_agg = _make_sc_agg()
_sc_deg = _make_sc_deg()


def _tc_layer1(x, aggp, degp, w_self, w_nei, b):
    def body(x_ref, aggp_ref, degp_ref, ws_ref, wn_ref, b_ref, o_ref):
        agg = aggp_ref[0] + aggp_ref[1]
        deg = jnp.maximum(degp_ref[0, :, 0:1] + degp_ref[1, :, 0:1], 1.0)
        h = jnp.dot(x_ref[...], ws_ref[...], preferred_element_type=jnp.float32)
        h += jnp.dot(agg / deg, wn_ref[...], preferred_element_type=jnp.float32)
        h += b_ref[...]
        o_ref[...] = jnp.maximum(h, 0.0)

    return pl.pallas_call(
        body,
        grid=(NN // R,),
        in_specs=[
            pl.BlockSpec((R, D), lambda i: (i, 0)),
            pl.BlockSpec((2, R, D), lambda i: (0, i, 0)),
            pl.BlockSpec((2, R, DEGW), lambda i: (0, i, 0)),
            pl.BlockSpec((D, D), lambda i: (0, 0)),
            pl.BlockSpec((D, D), lambda i: (0, 0)),
            pl.BlockSpec((1, D), lambda i: (0, 0)),
        ],
        out_specs=pl.BlockSpec((R, D), lambda i: (i, 0)),
        out_shape=jax.ShapeDtypeStruct((NN, D), jnp.float32),
    )(x, aggp, degp, w_self, w_nei, b)


def _tc_layer2_pool(h1, aggp, degp, w_self, w_nei, b, bat3, w_lat, b_lat):
    nb = NN // R

    def body(h_ref, aggp_ref, degp_ref, ws_ref, wn_ref, b_ref, bat_ref,
             wl_ref, bl_ref, z_ref, pool_acc, cnt_acc):
        i = pl.program_id(0)

        @pl.when(i == 0)
        def _():
            pool_acc[...] = jnp.zeros_like(pool_acc)
            cnt_acc[...] = jnp.zeros_like(cnt_acc)

        agg = aggp_ref[0] + aggp_ref[1]
        deg = jnp.maximum(degp_ref[0, :, 0:1] + degp_ref[1, :, 0:1], 1.0)
        h2 = jnp.dot(h_ref[...], ws_ref[...], preferred_element_type=jnp.float32)
        h2 += jnp.dot(agg / deg, wn_ref[...], preferred_element_type=jnp.float32)
        h2 += b_ref[...]
        h2 = jnp.maximum(h2, 0.0)

        ids = bat_ref[0, 0, :]
        rid = lax.broadcasted_iota(jnp.int32, (NG, R), 0)
        onehot = (rid == ids[None, :]).astype(jnp.float32)
        pool_acc[...] += jnp.dot(onehot, h2, preferred_element_type=jnp.float32)
        cnt_acc[...] += jnp.broadcast_to(
            jnp.sum(onehot, axis=1, keepdims=True), (NG, D))

        @pl.when(i == nb - 1)
        def _():
            pooled = pool_acc[...] / jnp.maximum(cnt_acc[...], 1.0)
            z_ref[...] = jnp.dot(pooled, wl_ref[...],
                                 preferred_element_type=jnp.float32) + bl_ref[...]

    return pl.pallas_call(
        body,
        grid=(nb,),
        in_specs=[
            pl.BlockSpec((R, D), lambda i: (i, 0)),
            pl.BlockSpec((2, R, D), lambda i: (0, i, 0)),
            pl.BlockSpec((2, R, DEGW), lambda i: (0, i, 0)),
            pl.BlockSpec((D, D), lambda i: (0, 0)),
            pl.BlockSpec((D, D), lambda i: (0, 0)),
            pl.BlockSpec((1, D), lambda i: (0, 0)),
            pl.BlockSpec((1, 1, R), lambda i: (i, 0, 0)),
            pl.BlockSpec((D, LAT), lambda i: (0, 0)),
            pl.BlockSpec((1, LAT), lambda i: (0, 0)),
        ],
        out_specs=pl.BlockSpec((NG, LAT), lambda i: (0, 0)),
        out_shape=jax.ShapeDtypeStruct((NG, LAT), jnp.float32),
        scratch_shapes=[
            pltpu.VMEM((NG, D), jnp.float32),
            pltpu.VMEM((NG, D), jnp.float32),
        ],
    )(h1, aggp, degp, w_self, w_nei, b, bat3, w_lat, b_lat)


def kernel(x, edge_index, batch, W1_self, W1_nei, b1, W2_self, W2_nei, b2,
           W_lat, b_lat):
    src = edge_index[0].astype(jnp.int32)
    dst = edge_index[1].astype(jnp.int32)
    pad = EP - EE
    srcf = jnp.concatenate([src, jnp.zeros((pad,), jnp.int32)])
    dstf = jnp.concatenate([dst, jnp.full((pad,), DUM, jnp.int32)])
    srcp = srcf.reshape(NGRP, CG)
    dstp = dstf.reshape(NGRP, CG)
    dstp_deg = dstf.reshape(NW, G, C)
    bat3 = batch.astype(jnp.int32).reshape(NN // R, 1, R)

    (degp,) = _sc_deg(dstp_deg)
    (agg1,) = _sc_agg(x, srcp, dstp)
    h1 = _tc_layer1(x, agg1, degp, W1_self, W1_nei, b1.reshape(1, D))
    (agg2,) = _sc_agg(h1, srcp, dstp)
    z = _tc_layer2_pool(h1, agg2, degp, W2_self, W2_nei, b2.reshape(1, D),
                        bat3, W_lat, b_lat.reshape(1, LAT))
    return z


# speedup vs baseline: 1.4437x; 1.1301x over previous
import functools

import jax
import jax.numpy as jnp
from jax import lax
from jax.experimental import pallas as pl
from jax.experimental.pallas import tpu as pltpu
from jax.experimental.pallas import tpu_sc as plsc

NN = 10000
EE = 320000
D = 128
LAT = 64
NG = 64
NW = 32
C = 128
G = 80
EP = NW * G * C
DUM = NN
NPAD = 10240
DEGW = 128
R = 1000


def _stripe_out(sid, cid, src_s, out_hbm):
    s_full = 640
    s_last = NN - 15 * s_full

    @pl.when(sid < 15)
    def _():
        pltpu.sync_copy(src_s.at[pl.ds(sid * s_full, s_full)],
                        out_hbm.at[cid, pl.ds(sid * s_full, s_full)])

    @pl.when(sid == 15)
    def _():
        pltpu.sync_copy(src_s.at[pl.ds(15 * s_full, s_last)],
                        out_hbm.at[cid, pl.ds(15 * s_full, s_last)])


CG = 128
NBUF = 2
G2 = 24
NGRP = EP // CG
N0 = 144
N1 = NGRP // 16 - N0


def _make_sc_agg():
    mesh = plsc.VectorSubcoreMesh(core_axis_name="c", subcore_axis_name="s")

    def body(x_hbm, src_hbm, dst_hbm, agg_out, src_v, dst_v, rows_v, agg_s,
             *sems):
        cid = lax.axis_index("c")
        sid = lax.axis_index("s")

        zv = jnp.zeros((16,), jnp.float32)
        for i in range(16):
            for j in range(D // 16):
                rows_v[0, i, pl.ds(j * 16, 16)] = zv
        zsrc = rows_v.at[0, pl.ds(0, 16)]

        rps = NPAD // 16

        @pl.loop(0, rps // 16)
        def _(k):
            pltpu.sync_copy(zsrc, agg_s.at[pl.ds(sid * rps + k * 16, 16)])

        plsc.subcore_barrier()

        def start(g, b):
            pltpu.async_copy(x_hbm.at[src_v.at[g]], rows_v.at[b], sems[b])

        def wait(b):
            pltpu.make_async_copy(x_hbm.at[src_v.at[0]], rows_v.at[b],
                                  sems[b]).wait()

        def scat(g, b):
            pltpu.sync_copy(rows_v.at[b], agg_s.at[dst_v.at[g]], add=True)

        def run_phase(gbase, glen):
            pltpu.sync_copy(src_hbm.at[pl.ds(gbase, glen)],
                            src_v.at[pl.ds(0, glen)])
            pltpu.sync_copy(dst_hbm.at[pl.ds(gbase, glen)],
                            dst_v.at[pl.ds(0, glen)])

            for b in range(NBUF):
                start(b, b)

            @pl.loop(0, (glen - NBUF) // NBUF)
            def _(i):
                g0 = i * NBUF
                for b in range(NBUF):
                    wait(b)
                    scat(g0 + b, b)
                    start(g0 + b + NBUF, b)

            for b in range(NBUF):
                wait(b)
                scat(glen - NBUF + b, b)

        @pl.when(cid == 0)
        def _():
            for ph in range(N0 // G2):
                run_phase(sid * N0 + ph * G2, G2)

        @pl.when(cid == 1)
        def _():
            run_phase(16 * N0 + sid * N1, N1)

        plsc.subcore_barrier()
        _stripe_out(sid, cid, agg_s, agg_out)

    return pl.kernel(
        body,
        out_type=[jax.ShapeDtypeStruct((2, NN, D), jnp.float32)],
        mesh=mesh,
        scratch_types=[
            pltpu.VMEM((G2, CG), jnp.int32),
            pltpu.VMEM((G2, CG), jnp.int32),
            pltpu.VMEM((NBUF, CG, D), jnp.float32),
            pltpu.VMEM_SHARED((NPAD, D), jnp.float32),
        ] + [pltpu.SemaphoreType.DMA] * NBUF)


def _make_sc_deg():
    mesh = plsc.VectorSubcoreMesh(core_axis_name="c", subcore_axis_name="s")

    def body(dst_hbm, deg_out, dst_v, ones_v, zdeg_v, deg_s):
        cid = lax.axis_index("c")
        sid = lax.axis_index("s")
        wid = sid * 2 + cid

        zv = jnp.zeros((16,), jnp.float32)
        ov = jnp.ones((16,), jnp.float32)
        for i in range(C):
            for j in range(DEGW // 16):
                ones_v[i, pl.ds(j * 16, 16)] = ov
        for i in range(16):
            for j in range(DEGW // 16):
                zdeg_v[i, pl.ds(j * 16, 16)] = zv

        rps = NPAD // 16

        @pl.loop(0, rps // 16)
        def _(k):
            pltpu.sync_copy(zdeg_v, deg_s.at[pl.ds(sid * rps + k * 16, 16)])

        plsc.subcore_barrier()
        pltpu.sync_copy(dst_hbm.at[wid], dst_v)

        @pl.loop(0, G)
        def _(g):
            pltpu.sync_copy(ones_v, deg_s.at[dst_v.at[g]], add=True)

        plsc.subcore_barrier()
        _stripe_out(sid, cid, deg_s, deg_out)

    return pl.kernel(
        body,
        out_type=[jax.ShapeDtypeStruct((2, NN, DEGW), jnp.float32)],
        mesh=mesh,
        scratch_types=[
            pltpu.VMEM((G, C), jnp.int32),
            pltpu.VMEM((C, DEGW), jnp.float32),
            pltpu.VMEM((16, DEGW), jnp.float32),
            pltpu.VMEM_SHARED((NPAD, DEGW), jnp.float32),
        ])


_sc_agg = _make_sc_agg()
_sc_deg = _make_sc_deg()


def _tc_layer1(x, aggp, degp, w_self, w_nei, b):
    def body(x_ref, aggp_ref, degp_ref, ws_ref, wn_ref, b_ref, o_ref):
        agg = aggp_ref[0] + aggp_ref[1]
        deg = jnp.maximum(degp_ref[0, :, 0:1] + degp_ref[1, :, 0:1], 1.0)
        h = jnp.dot(x_ref[...], ws_ref[...], preferred_element_type=jnp.float32)
        h += jnp.dot(agg / deg, wn_ref[...], preferred_element_type=jnp.float32)
        h += b_ref[...]
        o_ref[...] = jnp.maximum(h, 0.0)

    return pl.pallas_call(
        body,
        grid=(NN // R,),
        in_specs=[
            pl.BlockSpec((R, D), lambda i: (i, 0)),
            pl.BlockSpec((2, R, D), lambda i: (0, i, 0)),
            pl.BlockSpec((2, R, DEGW), lambda i: (0, i, 0)),
            pl.BlockSpec((D, D), lambda i: (0, 0)),
            pl.BlockSpec((D, D), lambda i: (0, 0)),
            pl.BlockSpec((1, D), lambda i: (0, 0)),
        ],
        out_specs=pl.BlockSpec((R, D), lambda i: (i, 0)),
        out_shape=jax.ShapeDtypeStruct((NN, D), jnp.float32),
    )(x, aggp, degp, w_self, w_nei, b)


def _tc_layer2_pool(h1, aggp, degp, w_self, w_nei, b, bat3, w_lat, b_lat):
    nb = NN // R

    def body(h_ref, aggp_ref, degp_ref, ws_ref, wn_ref, b_ref, bat_ref,
             wl_ref, bl_ref, z_ref, pool_acc, cnt_acc):
        i = pl.program_id(0)

        @pl.when(i == 0)
        def _():
            pool_acc[...] = jnp.zeros_like(pool_acc)
            cnt_acc[...] = jnp.zeros_like(cnt_acc)

        agg = aggp_ref[0] + aggp_ref[1]
        deg = jnp.maximum(degp_ref[0, :, 0:1] + degp_ref[1, :, 0:1], 1.0)
        h2 = jnp.dot(h_ref[...], ws_ref[...], preferred_element_type=jnp.float32)
        h2 += jnp.dot(agg / deg, wn_ref[...], preferred_element_type=jnp.float32)
        h2 += b_ref[...]
        h2 = jnp.maximum(h2, 0.0)

        ids = bat_ref[0, 0, :]
        rid = lax.broadcasted_iota(jnp.int32, (NG, R), 0)
        onehot = (rid == ids[None, :]).astype(jnp.float32)
        pool_acc[...] += jnp.dot(onehot, h2, preferred_element_type=jnp.float32)
        cnt_acc[...] += jnp.broadcast_to(
            jnp.sum(onehot, axis=1, keepdims=True), (NG, D))

        @pl.when(i == nb - 1)
        def _():
            pooled = pool_acc[...] / jnp.maximum(cnt_acc[...], 1.0)
            z_ref[...] = jnp.dot(pooled, wl_ref[...],
                                 preferred_element_type=jnp.float32) + bl_ref[...]

    return pl.pallas_call(
        body,
        grid=(nb,),
        in_specs=[
            pl.BlockSpec((R, D), lambda i: (i, 0)),
            pl.BlockSpec((2, R, D), lambda i: (0, i, 0)),
            pl.BlockSpec((2, R, DEGW), lambda i: (0, i, 0)),
            pl.BlockSpec((D, D), lambda i: (0, 0)),
            pl.BlockSpec((D, D), lambda i: (0, 0)),
            pl.BlockSpec((1, D), lambda i: (0, 0)),
            pl.BlockSpec((1, 1, R), lambda i: (i, 0, 0)),
            pl.BlockSpec((D, LAT), lambda i: (0, 0)),
            pl.BlockSpec((1, LAT), lambda i: (0, 0)),
        ],
        out_specs=pl.BlockSpec((NG, LAT), lambda i: (0, 0)),
        out_shape=jax.ShapeDtypeStruct((NG, LAT), jnp.float32),
        scratch_shapes=[
            pltpu.VMEM((NG, D), jnp.float32),
            pltpu.VMEM((NG, D), jnp.float32),
        ],
    )(h1, aggp, degp, w_self, w_nei, b, bat3, w_lat, b_lat)


def kernel(x, edge_index, batch, W1_self, W1_nei, b1, W2_self, W2_nei, b2,
           W_lat, b_lat):
    src = edge_index[0].astype(jnp.int32)
    dst = edge_index[1].astype(jnp.int32)
    pad = EP - EE
    srcf = jnp.concatenate([src, jnp.zeros((pad,), jnp.int32)])
    dstf = jnp.concatenate([dst, jnp.full((pad,), DUM, jnp.int32)])
    srcp = srcf.reshape(NGRP, CG)
    dstp = dstf.reshape(NGRP, CG)
    dstp_deg = dstf.reshape(NW, G, C)
    bat3 = batch.astype(jnp.int32).reshape(NN // R, 1, R)

    (degp,) = _sc_deg(dstp_deg)
    (agg1,) = _sc_agg(x, srcp, dstp)
    h1 = _tc_layer1(x, agg1, degp, W1_self, W1_nei, b1.reshape(1, D))
    (agg2,) = _sc_agg(h1, srcp, dstp)
    z = _tc_layer2_pool(h1, agg2, degp, W2_self, W2_nei, b2.reshape(1, D),
                        bat3, W_lat, b_lat.reshape(1, LAT))
    return z


# speedup vs baseline: 1.4551x; 1.0079x over previous
import functools

import jax
import jax.numpy as jnp
from jax import lax
from jax.experimental import pallas as pl
from jax.experimental.pallas import tpu as pltpu
from jax.experimental.pallas import tpu_sc as plsc

NN = 10000
EE = 320000
D = 128
LAT = 64
NG = 64
NW = 32
C = 128
G = 80
EP = NW * G * C
DUM = NN
NPAD = 10240
DEGW = 128
R = 1000


def _stripe_out(sid, cid, src_s, out_hbm):
    s_full = 640
    s_last = NN - 15 * s_full

    @pl.when(sid < 15)
    def _():
        pltpu.sync_copy(src_s.at[pl.ds(sid * s_full, s_full)],
                        out_hbm.at[cid, pl.ds(sid * s_full, s_full)])

    @pl.when(sid == 15)
    def _():
        pltpu.sync_copy(src_s.at[pl.ds(15 * s_full, s_last)],
                        out_hbm.at[cid, pl.ds(15 * s_full, s_last)])


CG = 128
NBUF = 2
G2 = 24
NGRP = EP // CG
N0 = 152
N1 = NGRP // 16 - N0


def _make_sc_agg():
    mesh = plsc.VectorSubcoreMesh(core_axis_name="c", subcore_axis_name="s")

    def body(x_hbm, src_hbm, dst_hbm, agg_out, src_v, dst_v, rows_v, agg_s,
             *sems):
        cid = lax.axis_index("c")
        sid = lax.axis_index("s")

        zv = jnp.zeros((16,), jnp.float32)
        for i in range(16):
            for j in range(D // 16):
                rows_v[0, i, pl.ds(j * 16, 16)] = zv
        zsrc = rows_v.at[0, pl.ds(0, 16)]

        rps = NPAD // 16

        @pl.loop(0, rps // 16)
        def _(k):
            pltpu.sync_copy(zsrc, agg_s.at[pl.ds(sid * rps + k * 16, 16)])

        plsc.subcore_barrier()

        def start(g, b):
            pltpu.async_copy(x_hbm.at[src_v.at[g]], rows_v.at[b], sems[b])

        def wait(b):
            pltpu.make_async_copy(x_hbm.at[src_v.at[0]], rows_v.at[b],
                                  sems[b]).wait()

        def scat(g, b):
            pltpu.sync_copy(rows_v.at[b], agg_s.at[dst_v.at[g]], add=True)

        def run_phase(gbase, glen):
            pltpu.sync_copy(src_hbm.at[pl.ds(gbase, glen)],
                            src_v.at[pl.ds(0, glen)])
            pltpu.sync_copy(dst_hbm.at[pl.ds(gbase, glen)],
                            dst_v.at[pl.ds(0, glen)])

            for b in range(NBUF):
                start(b, b)

            @pl.loop(0, (glen - NBUF) // NBUF)
            def _(i):
                g0 = i * NBUF
                for b in range(NBUF):
                    wait(b)
                    scat(g0 + b, b)
                    start(g0 + b + NBUF, b)

            for b in range(NBUF):
                wait(b)
                scat(glen - NBUF + b, b)

        @pl.when(cid == 0)
        def _():
            for ph in range(N0 // G2):
                run_phase(sid * N0 + ph * G2, G2)
            if N0 % G2:
                run_phase(sid * N0 + (N0 // G2) * G2, N0 % G2)

        @pl.when(cid == 1)
        def _():
            run_phase(16 * N0 + sid * N1, N1)

        plsc.subcore_barrier()
        _stripe_out(sid, cid, agg_s, agg_out)

    return pl.kernel(
        body,
        out_type=[jax.ShapeDtypeStruct((2, NN, D), jnp.float32)],
        mesh=mesh,
        scratch_types=[
            pltpu.VMEM((G2, CG), jnp.int32),
            pltpu.VMEM((G2, CG), jnp.int32),
            pltpu.VMEM((NBUF, CG, D), jnp.float32),
            pltpu.VMEM_SHARED((NPAD, D), jnp.float32),
        ] + [pltpu.SemaphoreType.DMA] * NBUF)


def _make_sc_deg():
    mesh = plsc.VectorSubcoreMesh(core_axis_name="c", subcore_axis_name="s")

    def body(dst_hbm, deg_out, dst_v, ones_v, zdeg_v, deg_s):
        cid = lax.axis_index("c")
        sid = lax.axis_index("s")
        wid = sid * 2 + cid

        zv = jnp.zeros((16,), jnp.float32)
        ov = jnp.ones((16,), jnp.float32)
        for i in range(C):
            for j in range(DEGW // 16):
                ones_v[i, pl.ds(j * 16, 16)] = ov
        for i in range(16):
            for j in range(DEGW // 16):
                zdeg_v[i, pl.ds(j * 16, 16)] = zv

        rps = NPAD // 16

        @pl.loop(0, rps // 16)
        def _(k):
            pltpu.sync_copy(zdeg_v, deg_s.at[pl.ds(sid * rps + k * 16, 16)])

        plsc.subcore_barrier()
        pltpu.sync_copy(dst_hbm.at[wid], dst_v)

        @pl.loop(0, G)
        def _(g):
            pltpu.sync_copy(ones_v, deg_s.at[dst_v.at[g]], add=True)

        plsc.subcore_barrier()
        _stripe_out(sid, cid, deg_s, deg_out)

    return pl.kernel(
        body,
        out_type=[jax.ShapeDtypeStruct((2, NN, DEGW), jnp.float32)],
        mesh=mesh,
        scratch_types=[
            pltpu.VMEM((G, C), jnp.int32),
            pltpu.VMEM((C, DEGW), jnp.float32),
            pltpu.VMEM((16, DEGW), jnp.float32),
            pltpu.VMEM_SHARED((NPAD, DEGW), jnp.float32),
        ])


_sc_agg = _make_sc_agg()
_sc_deg = _make_sc_deg()


def _tc_layer1(x, aggp, degp, w_self, w_nei, b):
    def body(x_ref, aggp_ref, degp_ref, ws_ref, wn_ref, b_ref, o_ref):
        agg = aggp_ref[0] + aggp_ref[1]
        deg = jnp.maximum(degp_ref[0, :, 0:1] + degp_ref[1, :, 0:1], 1.0)
        h = jnp.dot(x_ref[...], ws_ref[...], preferred_element_type=jnp.float32)
        h += jnp.dot(agg / deg, wn_ref[...], preferred_element_type=jnp.float32)
        h += b_ref[...]
        o_ref[...] = jnp.maximum(h, 0.0)

    return pl.pallas_call(
        body,
        grid=(NN // R,),
        in_specs=[
            pl.BlockSpec((R, D), lambda i: (i, 0)),
            pl.BlockSpec((2, R, D), lambda i: (0, i, 0)),
            pl.BlockSpec((2, R, DEGW), lambda i: (0, i, 0)),
            pl.BlockSpec((D, D), lambda i: (0, 0)),
            pl.BlockSpec((D, D), lambda i: (0, 0)),
            pl.BlockSpec((1, D), lambda i: (0, 0)),
        ],
        out_specs=pl.BlockSpec((R, D), lambda i: (i, 0)),
        out_shape=jax.ShapeDtypeStruct((NN, D), jnp.float32),
    )(x, aggp, degp, w_self, w_nei, b)


def _tc_layer2_pool(h1, aggp, degp, w_self, w_nei, b, bat3, w_lat, b_lat):
    nb = NN // R

    def body(h_ref, aggp_ref, degp_ref, ws_ref, wn_ref, b_ref, bat_ref,
             wl_ref, bl_ref, z_ref, pool_acc, cnt_acc):
        i = pl.program_id(0)

        @pl.when(i == 0)
        def _():
            pool_acc[...] = jnp.zeros_like(pool_acc)
            cnt_acc[...] = jnp.zeros_like(cnt_acc)

        agg = aggp_ref[0] + aggp_ref[1]
        deg = jnp.maximum(degp_ref[0, :, 0:1] + degp_ref[1, :, 0:1], 1.0)
        h2 = jnp.dot(h_ref[...], ws_ref[...], preferred_element_type=jnp.float32)
        h2 += jnp.dot(agg / deg, wn_ref[...], preferred_element_type=jnp.float32)
        h2 += b_ref[...]
        h2 = jnp.maximum(h2, 0.0)

        ids = bat_ref[0, 0, :]
        rid = lax.broadcasted_iota(jnp.int32, (NG, R), 0)
        onehot = (rid == ids[None, :]).astype(jnp.float32)
        pool_acc[...] += jnp.dot(onehot, h2, preferred_element_type=jnp.float32)
        cnt_acc[...] += jnp.broadcast_to(
            jnp.sum(onehot, axis=1, keepdims=True), (NG, D))

        @pl.when(i == nb - 1)
        def _():
            pooled = pool_acc[...] / jnp.maximum(cnt_acc[...], 1.0)
            z_ref[...] = jnp.dot(pooled, wl_ref[...],
                                 preferred_element_type=jnp.float32) + bl_ref[...]

    return pl.pallas_call(
        body,
        grid=(nb,),
        in_specs=[
            pl.BlockSpec((R, D), lambda i: (i, 0)),
            pl.BlockSpec((2, R, D), lambda i: (0, i, 0)),
            pl.BlockSpec((2, R, DEGW), lambda i: (0, i, 0)),
            pl.BlockSpec((D, D), lambda i: (0, 0)),
            pl.BlockSpec((D, D), lambda i: (0, 0)),
            pl.BlockSpec((1, D), lambda i: (0, 0)),
            pl.BlockSpec((1, 1, R), lambda i: (i, 0, 0)),
            pl.BlockSpec((D, LAT), lambda i: (0, 0)),
            pl.BlockSpec((1, LAT), lambda i: (0, 0)),
        ],
        out_specs=pl.BlockSpec((NG, LAT), lambda i: (0, 0)),
        out_shape=jax.ShapeDtypeStruct((NG, LAT), jnp.float32),
        scratch_shapes=[
            pltpu.VMEM((NG, D), jnp.float32),
            pltpu.VMEM((NG, D), jnp.float32),
        ],
    )(h1, aggp, degp, w_self, w_nei, b, bat3, w_lat, b_lat)


def kernel(x, edge_index, batch, W1_self, W1_nei, b1, W2_self, W2_nei, b2,
           W_lat, b_lat):
    src = edge_index[0].astype(jnp.int32)
    dst = edge_index[1].astype(jnp.int32)
    pad = EP - EE
    srcf = jnp.concatenate([src, jnp.zeros((pad,), jnp.int32)])
    dstf = jnp.concatenate([dst, jnp.full((pad,), DUM, jnp.int32)])
    srcp = srcf.reshape(NGRP, CG)
    dstp = dstf.reshape(NGRP, CG)
    dstp_deg = dstf.reshape(NW, G, C)
    bat3 = batch.astype(jnp.int32).reshape(NN // R, 1, R)

    (degp,) = _sc_deg(dstp_deg)
    (agg1,) = _sc_agg(x, srcp, dstp)
    h1 = _tc_layer1(x, agg1, degp, W1_self, W1_nei, b1.reshape(1, D))
    (agg2,) = _sc_agg(h1, srcp, dstp)
    z = _tc_layer2_pool(h1, agg2, degp, W2_self, W2_nei, b2.reshape(1, D),
                        bat3, W_lat, b_lat.reshape(1, LAT))
    return z
